# Initial kernel scaffold; baseline (speedup 1.0000x reference)
import jax
import jax.numpy as jnp
from jax.experimental import pallas as pl


def kernel(resources, operations, need_edge_attr, need_edge_index, same_edge_index, W_self, W_res, W_op, att_self, att_op, att_res):
    raise NotImplementedError("write your pallas kernel here")


# speedup vs baseline: 4.3481x; 4.3481x over previous
import functools

import jax
import jax.numpy as jnp
from jax import lax
from jax.experimental import pallas as pl
from jax.experimental.pallas import tpu as pltpu
from jax.experimental.pallas import tpu_sc as plsc

N_RES = 10000
N_OPS = 50000
E = 320000
D = 128
D_OPF = 112
D_EDGE = 16
F32 = jnp.float32

NC, NS = 2, 16
NW = NC * NS
EPW = E // NW
GRP_A = 2000
NG_A = EPW // GRP_A
GRP_C = 2000
NG_C = EPW // GRP_C
CH = 80
NCH = GRP_C // CH
RB = 80
NRB = N_RES // RB

BLK1 = 1000
BLK2 = 1000


def _t1_body(ops_ref, wop_ref, attop_ref, op_out, s2_out):
    w1 = wop_ref[:D_OPF, :]
    o = jnp.dot(ops_ref[...], w1, preferred_element_type=F32)
    op_out[...] = o
    s2_out[...] = jnp.dot(o, attop_ref[D:, :], preferred_element_type=F32)


def _t2_body(res_ref, wself_ref, wres_ref, amat_ref, bmat_ref, wop_ref,
             attop_ref, sr_out, rp_out, scal_out, v_out):
    r = res_ref[...]
    sr = jnp.dot(r, wself_ref[...], preferred_element_type=F32)
    rp = jnp.dot(r, wres_ref[...], preferred_element_type=F32)
    sr_out[...] = sr
    rp_out[...] = rp
    scal_out[...] = (jnp.dot(sr, amat_ref[...], preferred_element_type=F32)
                     + jnp.dot(rp, bmat_ref[...], preferred_element_type=F32))
    v_out[...] = jnp.dot(wop_ref[D_OPF:, :], attop_ref[D:, :],
                         preferred_element_type=F32)


def _t3_body(attr2_ref, vkron_ref, c_out):
    c_out[...] = jnp.dot(attr2_ref[...], vkron_ref[...],
                         preferred_element_type=F32)


def _max_body(ln_ref, ls_ref, scal_ref, m_out):
    a = jnp.max(ln_ref[...])
    b = jnp.max(ls_ref[...])
    s3 = scal_ref[:, 3:4]
    c = jnp.max(jnp.where(s3 >= 0, s3, 0.2 * s3))
    bm = jnp.maximum(jnp.maximum(a, b), c)
    m_out[...] = jnp.full((1, 16), bm, F32)


def _fin_body(sr_ref, scal_ref, acc0_ref, acc1_ref, a0_ref, a1_ref, wop_ref,
              m_ref, zp_ref, scalfull_ref, out_ref):
    m = m_ref[0, 0]
    sf = scalfull_ref[:, 3:4]
    lf = jnp.where(sf >= 0, sf, 0.2 * sf)
    z = jnp.sum(jnp.exp(lf - m)) + jnp.sum(zp_ref[...])
    s3 = scal_ref[:, 3:4]
    l3 = jnp.where(s3 >= 0, s3, 0.2 * s3)
    wself = jnp.exp(l3 - m)
    a16 = a0_ref[:, :D_EDGE] + a1_ref[:, :D_EDGE]
    w2 = wop_ref[D_OPF:, :]
    pre = (wself * sr_ref[...] + acc0_ref[...] + acc1_ref[...]
           + jnp.dot(a16, w2, preferred_element_type=F32))
    x = pre / z
    out_ref[...] = jnp.where(x > 0, x, jnp.exp(x) - 1.0)


def _sc_mesh():
    return plsc.VectorSubcoreMesh(core_axis_name="c", subcore_axis_name="s",
                                  num_cores=NC, num_subcores=NS)


@functools.partial(
    pl.kernel,
    out_type=[jax.ShapeDtypeStruct((E,), F32),
              jax.ShapeDtypeStruct((E,), F32)],
    mesh=_sc_mesh(),
    compiler_params=pltpu.CompilerParams(needs_layout_passes=False),
    scratch_types=[
        pltpu.VMEM((4 * N_RES,), F32),
        pltpu.VMEM((N_OPS,), F32),
        pltpu.VMEM((GRP_A,), jnp.int32),
        pltpu.VMEM((GRP_A,), jnp.int32),
        pltpu.VMEM((GRP_A,), F32),
        pltpu.VMEM((GRP_A,), F32),
    ],
)
def _logits_kernel(scal_h, s2_h, c_h, ns_h, nd_h, ss_h, sd_h,
                   lneed_h, lsame_h,
                   scal_v, s2_v, src_v, dst_v, c_v, l_v):
    cid = lax.axis_index("c")
    sid = lax.axis_index("s")
    wid = sid * NC + cid
    base = wid * EPW
    pltpu.sync_copy(scal_h, scal_v)
    pltpu.sync_copy(s2_h, s2_v)

    def need_group(g, carry):
        gb = base + g * GRP_A
        pltpu.sync_copy(ns_h.at[pl.ds(gb, GRP_A)], src_v)
        pltpu.sync_copy(nd_h.at[pl.ds(gb, GRP_A)], dst_v)
        pltpu.sync_copy(c_h.at[pl.ds(gb, GRP_A)], c_v)

        def tile16(t, carry2):
            s16 = src_v[pl.ds(t * 16, 16)]
            d16 = dst_v[pl.ds(t * 16, 16)]
            g1 = plsc.load_gather(scal_v, [d16 * 4])
            g2 = plsc.load_gather(s2_v, [s16])
            x = g1 + g2 + c_v[pl.ds(t * 16, 16)]
            l_v[pl.ds(t * 16, 16)] = jnp.where(x >= 0, x, 0.2 * x)
            return carry2

        lax.fori_loop(0, GRP_A // 16, tile16, 0)
        pltpu.sync_copy(l_v, lneed_h.at[pl.ds(gb, GRP_A)])
        return carry

    lax.fori_loop(0, NG_A, need_group, 0)

    def same_group(g, carry):
        gb = base + g * GRP_A
        pltpu.sync_copy(ss_h.at[pl.ds(gb, GRP_A)], src_v)
        pltpu.sync_copy(sd_h.at[pl.ds(gb, GRP_A)], dst_v)

        def tile16(t, carry2):
            s16 = src_v[pl.ds(t * 16, 16)]
            d16 = dst_v[pl.ds(t * 16, 16)]
            g1 = plsc.load_gather(scal_v, [d16 * 4 + 1])
            g2 = plsc.load_gather(scal_v, [s16 * 4 + 2])
            x = g1 + g2
            l_v[pl.ds(t * 16, 16)] = jnp.where(x >= 0, x, 0.2 * x)
            return carry2

        lax.fori_loop(0, GRP_A // 16, tile16, 0)
        pltpu.sync_copy(l_v, lsame_h.at[pl.ds(gb, GRP_A)])
        return carry

    lax.fori_loop(0, NG_A, same_group, 0)


@functools.partial(
    pl.kernel,
    out_type=[jax.ShapeDtypeStruct((2 * N_RES, D), F32),
              jax.ShapeDtypeStruct((NW * 16,), F32)],
    mesh=_sc_mesh(),
    compiler_params=pltpu.CompilerParams(needs_layout_passes=False),
    scratch_types=[
        pltpu.VMEM_SHARED((N_RES, D), F32),
        pltpu.VMEM((GRP_C,), jnp.int32),
        pltpu.VMEM((GRP_C,), jnp.int32),
        pltpu.VMEM((GRP_C,), F32),
        pltpu.VMEM((GRP_C,), F32),
        pltpu.VMEM((CH, D), F32),
        pltpu.VMEM((CH,), jnp.int32),
        pltpu.VMEM((CH,), jnp.int32),
        pltpu.VMEM((RB, D), F32),
        pltpu.VMEM((1, 16), F32),
        pltpu.VMEM((16,), F32),
        pltpu.SemaphoreType.DMA,
    ],
)
def _heavy_kernel(op_h, rp_h, ln_h, ls_h, ns_h, nd_h, ss_h, sd_h, m_h,
                  acc_out, z_out,
                  acc_sh, srcb, dstb, lb, wb, rows,
                  srcch, dstch, zb, m_v, zsum_v, sem):
    cid = lax.axis_index("c")
    sid = lax.axis_index("s")
    wid = sid * NC + cid
    base = wid * EPW
    zeros16 = jnp.zeros((16,), F32)

    def zrow(i, c):
        zb[i // 8, pl.ds((i % 8) * 16, 16)] = zeros16
        return c

    lax.fori_loop(0, RB * 8, zrow, 0)

    for t in range(8):
        b = sid + t * NS

        @pl.when(b < NRB)
        def _():
            pltpu.sync_copy(zb, acc_sh.at[pl.ds(b * RB, RB)])

    pltpu.sync_copy(m_h, m_v)
    plsc.subcore_barrier()

    m16 = m_v[0, :]

    def run_phase(table_h, s_h, d_h, l_h, zacc0):
        def group(g, zacc):
            gb = base + g * GRP_C
            pltpu.sync_copy(s_h.at[pl.ds(gb, GRP_C)], srcb)
            pltpu.sync_copy(d_h.at[pl.ds(gb, GRP_C)], dstb)
            pltpu.sync_copy(l_h.at[pl.ds(gb, GRP_C)], lb)

            def wcomp(t, za_):
                w16 = jnp.exp(lb[pl.ds(t * 16, 16)] - m16)
                wb[pl.ds(t * 16, 16)] = w16
                return za_ + w16

            zacc = lax.fori_loop(0, GRP_C // 16, wcomp, zacc)

            def chunk(ck, c):
                c0 = ck * CH

                def cpidx(t, c2):
                    srcch[pl.ds(t * 16, 16)] = srcb[pl.ds(c0 + t * 16, 16)]
                    dstch[pl.ds(t * 16, 16)] = dstb[pl.ds(c0 + t * 16, 16)]
                    return c2

                lax.fori_loop(0, CH // 16, cpidx, 0)
                pltpu.async_copy(table_h.at[srcch], rows, sem).wait()

                def scale(e, c2):
                    wspl = plsc.load_gather(
                        wb, [jnp.full((16,), c0 + e, jnp.int32)])
                    for k in range(D // 16):
                        rows[e, pl.ds(k * 16, 16)] = (
                            rows[e, pl.ds(k * 16, 16)] * wspl)
                    return c2

                lax.fori_loop(0, CH, scale, 0)
                pltpu.sync_copy(rows, acc_sh.at[dstch], add=True)
                return c

            lax.fori_loop(0, NCH, chunk, 0)
            return zacc

        return lax.fori_loop(0, NG_C, group, zacc0)

    zacc = jnp.zeros((16,), F32)
    zacc = run_phase(op_h, ns_h, nd_h, ln_h, zacc)
    zacc = run_phase(rp_h, ss_h, sd_h, ls_h, zacc)
    zsum_v[...] = zacc
    pltpu.sync_copy(zsum_v, z_out.at[pl.ds(wid * 16, 16)])
    plsc.subcore_barrier()

    for t in range(8):
        b = sid + t * NS

        @pl.when(b < NRB)
        def _():
            pltpu.sync_copy(acc_sh.at[pl.ds(b * RB, RB)],
                            acc_out.at[pl.ds(cid * N_RES + b * RB, RB)])


HA = N_RES // 2
JROW = HA
RBA = 40
NBA = HA // RBA
NZA = NBA + 1


@functools.partial(
    pl.kernel,
    out_type=jax.ShapeDtypeStruct((2 * N_RES, D), F32),
    mesh=_sc_mesh(),
    compiler_params=pltpu.CompilerParams(needs_layout_passes=False),
    scratch_types=[
        pltpu.VMEM_SHARED((HA + RBA, D), F32),
        pltpu.VMEM((GRP_C,), jnp.int32),
        pltpu.VMEM((GRP_C,), F32),
        pltpu.VMEM((GRP_C,), F32),
        pltpu.VMEM((GRP_C * D_EDGE,), F32),
        pltpu.VMEM((CH, D), F32),
        pltpu.VMEM((CH,), jnp.int32),
        pltpu.VMEM((RBA, D), F32),
        pltpu.VMEM((1, 16), F32),
    ],
)
def _attr_kernel(ln_h, nd_h, attr_h, m_h, a16_out,
                 a16_sh, dstb, lb, wb, attrb, attr_sc, dstch, zb, m_v):
    cid = lax.axis_index("c")
    sid = lax.axis_index("s")
    wid = sid * NC + cid
    base = wid * EPW
    zeros16 = jnp.zeros((16,), F32)

    def zrow(i, c):
        zb[i // 8, pl.ds((i % 8) * 16, 16)] = zeros16
        return c

    lax.fori_loop(0, RBA * 8, zrow, 0)

    def zrow2(i, c):
        attr_sc[i // 8, pl.ds((i % 8) * 16, 16)] = zeros16
        return c

    lax.fori_loop(0, CH * 8, zrow2, 0)

    pltpu.sync_copy(m_h, m_v)
    m16 = m_v[0, :]

    for p in range(2):
        lo = p * HA

        for t in range(8):
            b = sid + t * NS

            @pl.when(b < NZA)
            def _():
                pltpu.sync_copy(zb, a16_sh.at[pl.ds(b * RBA, RBA)])

        plsc.subcore_barrier()

        def group(g, c):
            gb = base + g * GRP_C
            pltpu.sync_copy(nd_h.at[pl.ds(gb, GRP_C)], dstb)
            pltpu.sync_copy(ln_h.at[pl.ds(gb, GRP_C)], lb)
            pltpu.sync_copy(attr_h.at[pl.ds(gb * D_EDGE, GRP_C * D_EDGE)],
                            attrb)

            def wcomp(t, c2):
                wb[pl.ds(t * 16, 16)] = jnp.exp(lb[pl.ds(t * 16, 16)] - m16)
                return c2

            lax.fori_loop(0, GRP_C // 16, wcomp, 0)

            def chunk(ck, c2):
                c0 = ck * CH

                def cpidx(t, c3):
                    d16 = dstb[pl.ds(c0 + t * 16, 16)] - lo
                    ok = jnp.logical_and(d16 >= 0, d16 < HA)
                    dstch[pl.ds(t * 16, 16)] = jnp.where(ok, d16, JROW)
                    return c3

                lax.fori_loop(0, CH // 16, cpidx, 0)

                def scale(e, c3):
                    wspl = plsc.load_gather(
                        wb, [jnp.full((16,), c0 + e, jnp.int32)])
                    attr_sc[e, pl.ds(0, D_EDGE)] = (
                        attrb[pl.ds((c0 + e) * D_EDGE, D_EDGE)] * wspl)
                    return c3

                lax.fori_loop(0, CH, scale, 0)
                pltpu.sync_copy(attr_sc, a16_sh.at[dstch], add=True)
                return c2

            lax.fori_loop(0, NCH, chunk, 0)
            return c

        lax.fori_loop(0, NG_C, group, 0)
        plsc.subcore_barrier()

        for t in range(8):
            b = sid + t * NS

            @pl.when(b < NBA)
            def _():
                pltpu.sync_copy(
                    a16_sh.at[pl.ds(b * RBA, RBA)],
                    a16_out.at[pl.ds(cid * N_RES + lo + b * RBA, RBA)])

        plsc.subcore_barrier()


def kernel(resources, operations, need_edge_attr, need_edge_index,
           same_edge_index, W_self, W_res, W_op, att_self, att_op, att_res):
    ns = need_edge_index[0].astype(jnp.int32)
    nd = need_edge_index[1].astype(jnp.int32)
    ss = same_edge_index[0].astype(jnp.int32)
    sd = same_edge_index[1].astype(jnp.int32)

    a_mat = jnp.concatenate(
        [att_op[:D], att_res[:D], jnp.zeros((D, 1), F32),
         att_self[:D] + att_self[D:]], axis=1)
    b_mat = jnp.concatenate(
        [jnp.zeros((D, 2), F32), att_res[D:], jnp.zeros((D, 1), F32)], axis=1)
    attr_flat = need_edge_attr.reshape(-1)
    attr2 = jnp.pad(attr_flat, (0, 60 * 128 * D_EDGE)).reshape(2560,
                                                              128 * D_EDGE)

    op_proj, s2col = pl.pallas_call(
        _t1_body,
        grid=(N_OPS // BLK1,),
        in_specs=[pl.BlockSpec((BLK1, D_OPF), lambda i: (i, 0)),
                  pl.BlockSpec((D_OPF + D_EDGE, D), lambda i: (0, 0)),
                  pl.BlockSpec((2 * D, 1), lambda i: (0, 0))],
        out_specs=[pl.BlockSpec((BLK1, D), lambda i: (i, 0)),
                   pl.BlockSpec((BLK1, 1), lambda i: (i, 0))],
        out_shape=[jax.ShapeDtypeStruct((N_OPS, D), F32),
                   jax.ShapeDtypeStruct((N_OPS, 1), F32)],
    )(operations, W_op, att_op)

    sr, rp, scal, v2 = pl.pallas_call(
        _t2_body,
        grid=(N_RES // BLK2,),
        in_specs=[pl.BlockSpec((BLK2, D), lambda i: (i, 0)),
                  pl.BlockSpec((D, D), lambda i: (0, 0)),
                  pl.BlockSpec((D, D), lambda i: (0, 0)),
                  pl.BlockSpec((D, 4), lambda i: (0, 0)),
                  pl.BlockSpec((D, 4), lambda i: (0, 0)),
                  pl.BlockSpec((D_OPF + D_EDGE, D), lambda i: (0, 0)),
                  pl.BlockSpec((2 * D, 1), lambda i: (0, 0))],
        out_specs=[pl.BlockSpec((BLK2, D), lambda i: (i, 0)),
                   pl.BlockSpec((BLK2, D), lambda i: (i, 0)),
                   pl.BlockSpec((BLK2, 4), lambda i: (i, 0)),
                   pl.BlockSpec((D_EDGE, 1), lambda i: (0, 0))],
        out_shape=[jax.ShapeDtypeStruct((N_RES, D), F32),
                   jax.ShapeDtypeStruct((N_RES, D), F32),
                   jax.ShapeDtypeStruct((N_RES, 4), F32),
                   jax.ShapeDtypeStruct((D_EDGE, 1), F32)],
    )(resources, W_self, W_res, a_mat, b_mat, W_op, att_op)

    s2f = s2col.reshape(-1)
    scal_flat = scal.reshape(-1)

    vkron = jnp.kron(jnp.eye(128, dtype=F32), v2)
    c2d = pl.pallas_call(
        _t3_body,
        grid=(5,),
        in_specs=[pl.BlockSpec((512, 128 * D_EDGE), lambda i: (i, 0)),
                  pl.BlockSpec((128 * D_EDGE, 128), lambda i: (0, 0))],
        out_specs=pl.BlockSpec((512, 128), lambda i: (i, 0)),
        out_shape=jax.ShapeDtypeStruct((2560, 128), F32),
    )(attr2, vkron)
    c_flat = c2d.reshape(-1)

    lneed, lsame = _logits_kernel(scal_flat, s2f, c_flat, ns, nd, ss, sd)

    m = pl.pallas_call(
        _max_body,
        out_shape=jax.ShapeDtypeStruct((1, 16), F32),
    )(lneed.reshape(E // 128, 128), lsame.reshape(E // 128, 128), scal)

    accp, zpf = _heavy_kernel(op_proj, rp, lneed, lsame, ns, nd, ss, sd, m)
    a16p = _attr_kernel(lneed, nd, attr_flat, m)
    zp = zpf.reshape(NW, 16)

    emb = pl.pallas_call(
        _fin_body,
        grid=(N_RES // BLK2,),
        in_specs=[pl.BlockSpec((BLK2, D), lambda i: (i, 0)),
                  pl.BlockSpec((BLK2, 4), lambda i: (i, 0)),
                  pl.BlockSpec((BLK2, D), lambda i: (i, 0)),
                  pl.BlockSpec((BLK2, D), lambda i: (i + 10, 0)),
                  pl.BlockSpec((BLK2, D), lambda i: (i, 0)),
                  pl.BlockSpec((BLK2, D), lambda i: (i + 10, 0)),
                  pl.BlockSpec((D_OPF + D_EDGE, D), lambda i: (0, 0)),
                  pl.BlockSpec((1, 16), lambda i: (0, 0)),
                  pl.BlockSpec((NW, 16), lambda i: (0, 0)),
                  pl.BlockSpec((N_RES, 4), lambda i: (0, 0))],
        out_specs=pl.BlockSpec((BLK2, D), lambda i: (i, 0)),
        out_shape=jax.ShapeDtypeStruct((N_RES, D), F32),
    )(sr, scal, accp, accp, a16p, a16p, W_op, m, zp, scal)

    return emb


# speedup vs baseline: 5.6845x; 1.3074x over previous
import functools

import jax
import jax.numpy as jnp
from jax import lax
from jax.experimental import pallas as pl
from jax.experimental.pallas import tpu as pltpu
from jax.experimental.pallas import tpu_sc as plsc

N_RES = 10000
N_OPS = 50000
E = 320000
D = 128
D_OPF = 112
D_EDGE = 16
F32 = jnp.float32

NC, NS = 2, 16
NW = NC * NS
EPW = E // NW
GRP_A = 2000
NG_A = EPW // GRP_A
GRP_C = 2000
NG_C = EPW // GRP_C
CH = 80
NCH = GRP_C // CH
RB = 80
NRB = N_RES // RB

BLK1 = 1000
BLK2 = 1000


def _t1_body(ops_ref, wop_ref, attop_ref, op_out, s2_out):
    w1 = wop_ref[:D_OPF, :]
    o = jnp.dot(ops_ref[...], w1, preferred_element_type=F32)
    op_out[...] = o
    s2_out[...] = jnp.dot(o, attop_ref[D:, :], preferred_element_type=F32)


def _t2_body(res_ref, wself_ref, wres_ref, amat_ref, bmat_ref, wop_ref,
             attop_ref, sr_out, rp_out, scal_out, v_out):
    r = res_ref[...]
    sr = jnp.dot(r, wself_ref[...], preferred_element_type=F32)
    rp = jnp.dot(r, wres_ref[...], preferred_element_type=F32)
    sr_out[...] = sr
    rp_out[...] = rp
    scal_out[...] = (jnp.dot(sr, amat_ref[...], preferred_element_type=F32)
                     + jnp.dot(rp, bmat_ref[...], preferred_element_type=F32))
    v_out[...] = jnp.dot(wop_ref[D_OPF:, :], attop_ref[D:, :],
                         preferred_element_type=F32)


def _t3_body(attr2_ref, vkron_ref, c_out):
    c_out[...] = jnp.dot(attr2_ref[...], vkron_ref[...],
                         preferred_element_type=F32)


def _max_body(ln_ref, ls_ref, scal_ref, m_out):
    a = jnp.max(ln_ref[...])
    b = jnp.max(ls_ref[...])
    s3 = scal_ref[:, 3:4]
    c = jnp.max(jnp.where(s3 >= 0, s3, 0.2 * s3))
    bm = jnp.maximum(jnp.maximum(a, b), c)
    m_out[...] = jnp.full((1, 16), bm, F32)


def _fin_body(sr_ref, scal_ref, acc0_ref, acc1_ref, a0_ref, a1_ref, wop_ref,
              m_ref, zp_ref, scalfull_ref, out_ref):
    m = m_ref[0, 0]
    sf = scalfull_ref[:, 3:4]
    lf = jnp.where(sf >= 0, sf, 0.2 * sf)
    z = jnp.sum(jnp.exp(lf - m)) + jnp.sum(zp_ref[...])
    s3 = scal_ref[:, 3:4]
    l3 = jnp.where(s3 >= 0, s3, 0.2 * s3)
    wself = jnp.exp(l3 - m)
    a16 = a0_ref[:, :D_EDGE] + a1_ref[:, :D_EDGE]
    w2 = wop_ref[D_OPF:, :]
    pre = (wself * sr_ref[...] + acc0_ref[...] + acc1_ref[...]
           + jnp.dot(a16, w2, preferred_element_type=F32))
    x = pre / z
    out_ref[...] = jnp.where(x > 0, x, jnp.exp(x) - 1.0)


def _sc_mesh():
    return plsc.VectorSubcoreMesh(core_axis_name="c", subcore_axis_name="s",
                                  num_cores=NC, num_subcores=NS)


@functools.partial(
    pl.kernel,
    out_type=[jax.ShapeDtypeStruct((E,), F32),
              jax.ShapeDtypeStruct((E,), F32)],
    mesh=_sc_mesh(),
    compiler_params=pltpu.CompilerParams(needs_layout_passes=False),
    scratch_types=[
        pltpu.VMEM((4 * N_RES,), F32),
        pltpu.VMEM((N_OPS,), F32),
        pltpu.VMEM((GRP_A,), jnp.int32),
        pltpu.VMEM((GRP_A,), jnp.int32),
        pltpu.VMEM((GRP_A,), F32),
        pltpu.VMEM((GRP_A,), F32),
    ],
)
def _logits_kernel(scal_h, s2_h, c_h, ns_h, nd_h, ss_h, sd_h,
                   lneed_h, lsame_h,
                   scal_v, s2_v, src_v, dst_v, c_v, l_v):
    cid = lax.axis_index("c")
    sid = lax.axis_index("s")
    wid = sid * NC + cid
    base = wid * EPW
    pltpu.sync_copy(scal_h, scal_v)
    pltpu.sync_copy(s2_h, s2_v)

    def need_group(g, carry):
        gb = base + g * GRP_A
        pltpu.sync_copy(ns_h.at[pl.ds(gb, GRP_A)], src_v)
        pltpu.sync_copy(nd_h.at[pl.ds(gb, GRP_A)], dst_v)
        pltpu.sync_copy(c_h.at[pl.ds(gb, GRP_A)], c_v)

        def tile16(t, carry2):
            s16 = src_v[pl.ds(t * 16, 16)]
            d16 = dst_v[pl.ds(t * 16, 16)]
            g1 = plsc.load_gather(scal_v, [d16 * 4])
            g2 = plsc.load_gather(s2_v, [s16])
            x = g1 + g2 + c_v[pl.ds(t * 16, 16)]
            l_v[pl.ds(t * 16, 16)] = jnp.where(x >= 0, x, 0.2 * x)
            return carry2

        lax.fori_loop(0, GRP_A // 16, tile16, 0)
        pltpu.sync_copy(l_v, lneed_h.at[pl.ds(gb, GRP_A)])
        return carry

    lax.fori_loop(0, NG_A, need_group, 0)

    def same_group(g, carry):
        gb = base + g * GRP_A
        pltpu.sync_copy(ss_h.at[pl.ds(gb, GRP_A)], src_v)
        pltpu.sync_copy(sd_h.at[pl.ds(gb, GRP_A)], dst_v)

        def tile16(t, carry2):
            s16 = src_v[pl.ds(t * 16, 16)]
            d16 = dst_v[pl.ds(t * 16, 16)]
            g1 = plsc.load_gather(scal_v, [d16 * 4 + 1])
            g2 = plsc.load_gather(scal_v, [s16 * 4 + 2])
            x = g1 + g2
            l_v[pl.ds(t * 16, 16)] = jnp.where(x >= 0, x, 0.2 * x)
            return carry2

        lax.fori_loop(0, GRP_A // 16, tile16, 0)
        pltpu.sync_copy(l_v, lsame_h.at[pl.ds(gb, GRP_A)])
        return carry

    lax.fori_loop(0, NG_A, same_group, 0)


@functools.partial(
    pl.kernel,
    out_type=[jax.ShapeDtypeStruct((2 * N_RES, D), F32),
              jax.ShapeDtypeStruct((NW * 16,), F32)],
    mesh=_sc_mesh(),
    compiler_params=pltpu.CompilerParams(needs_layout_passes=False),
    scratch_types=[
        pltpu.VMEM_SHARED((N_RES, D), F32),
        pltpu.VMEM((GRP_C,), jnp.int32),
        pltpu.VMEM((GRP_C,), jnp.int32),
        pltpu.VMEM((GRP_C,), F32),
        pltpu.VMEM((GRP_C,), F32),
        pltpu.VMEM((2, CH, D), F32),
        pltpu.VMEM((2, CH), jnp.int32),
        pltpu.VMEM((2, CH), jnp.int32),
        pltpu.VMEM((RB, D), F32),
        pltpu.VMEM((1, 16), F32),
        pltpu.VMEM((16,), F32),
        pltpu.SemaphoreType.DMA,
    ],
)
def _heavy_kernel(op_h, rp_h, ln_h, ls_h, ns_h, nd_h, ss_h, sd_h, m_h,
                  acc_out, z_out,
                  acc_sh, srcb, dstb, lb, wb, rows,
                  srcch, dstch, zb, m_v, zsum_v, sem):
    cid = lax.axis_index("c")
    sid = lax.axis_index("s")
    wid = sid * NC + cid
    base = wid * EPW
    zeros16 = jnp.zeros((16,), F32)

    def zrow(i, c):
        zb[i // 8, pl.ds((i % 8) * 16, 16)] = zeros16
        return c

    lax.fori_loop(0, RB * 8, zrow, 0)

    for t in range(8):
        b = sid + t * NS

        @pl.when(b < NRB)
        def _():
            pltpu.sync_copy(zb, acc_sh.at[pl.ds(b * RB, RB)])

    pltpu.sync_copy(m_h, m_v)
    plsc.subcore_barrier()

    m16 = m_v[0, :]

    def run_phase(table_h, s_h, d_h, l_h, zacc0):
        def group(g, zacc):
            gb = base + g * GRP_C
            pltpu.sync_copy(s_h.at[pl.ds(gb, GRP_C)], srcb)
            pltpu.sync_copy(d_h.at[pl.ds(gb, GRP_C)], dstb)
            pltpu.sync_copy(l_h.at[pl.ds(gb, GRP_C)], lb)

            def wcomp(t, za_):
                w16 = jnp.exp(lb[pl.ds(t * 16, 16)] - m16)
                wb[pl.ds(t * 16, 16)] = w16
                return za_ + w16

            zacc = lax.fori_loop(0, GRP_C // 16, wcomp, zacc)

            def build(ck, b):
                c0 = ck * CH

                def cpidx(t, c2):
                    srcch[b, pl.ds(t * 16, 16)] = srcb[pl.ds(c0 + t * 16, 16)]
                    dstch[b, pl.ds(t * 16, 16)] = dstb[pl.ds(c0 + t * 16, 16)]
                    return c2

                lax.fori_loop(0, CH // 16, cpidx, 0)

            def start(b):
                pltpu.async_copy(table_h.at[srcch.at[b]], rows.at[b], sem)

            def drain(b):
                pltpu.make_async_copy(table_h.at[srcch.at[b]], rows.at[b],
                                      sem).wait()

            def consume(ck, b):
                def scale(j, c2):
                    e = 2 * j
                    w0 = plsc.load_gather(
                        wb, [jnp.full((16,), ck * CH + e, jnp.int32)])
                    w1 = plsc.load_gather(
                        wb, [jnp.full((16,), ck * CH + e + 1, jnp.int32)])
                    for k in range(D // 16):
                        rows[b, e, pl.ds(k * 16, 16)] = (
                            rows[b, e, pl.ds(k * 16, 16)] * w0)
                    for k in range(D // 16):
                        rows[b, e + 1, pl.ds(k * 16, 16)] = (
                            rows[b, e + 1, pl.ds(k * 16, 16)] * w1)
                    return c2

                lax.fori_loop(0, CH // 2, scale, 0)
                pltpu.sync_copy(rows.at[b], acc_sh.at[dstch.at[b]], add=True)

            build(0, 0)
            start(0)

            def pair(i, c):
                ck0 = 2 * i
                build(ck0 + 1, 1)
                start(1)
                drain(0)
                consume(ck0, 0)
                build(ck0 + 2, 0)
                start(0)
                drain(1)
                consume(ck0 + 1, 1)
                return c

            lax.fori_loop(0, NCH // 2, pair, 0)
            drain(0)
            consume(NCH - 1, 0)
            return zacc

        return lax.fori_loop(0, NG_C, group, zacc0)

    zacc = jnp.zeros((16,), F32)
    zacc = run_phase(op_h, ns_h, nd_h, ln_h, zacc)
    zacc = run_phase(rp_h, ss_h, sd_h, ls_h, zacc)
    zsum_v[...] = zacc
    pltpu.sync_copy(zsum_v, z_out.at[pl.ds(wid * 16, 16)])
    plsc.subcore_barrier()

    for t in range(8):
        b = sid + t * NS

        @pl.when(b < NRB)
        def _():
            pltpu.sync_copy(acc_sh.at[pl.ds(b * RB, RB)],
                            acc_out.at[pl.ds(cid * N_RES + b * RB, RB)])


HA = N_RES // 2
JROW = HA
RBA = 40
NBA = HA // RBA
NZA = NBA + 1


@functools.partial(
    pl.kernel,
    out_type=jax.ShapeDtypeStruct((2 * N_RES, D), F32),
    mesh=_sc_mesh(),
    compiler_params=pltpu.CompilerParams(needs_layout_passes=False),
    scratch_types=[
        pltpu.VMEM_SHARED((HA + RBA, D), F32),
        pltpu.VMEM((GRP_C,), jnp.int32),
        pltpu.VMEM((GRP_C,), F32),
        pltpu.VMEM((GRP_C,), F32),
        pltpu.VMEM((GRP_C * D_EDGE,), F32),
        pltpu.VMEM((CH, D), F32),
        pltpu.VMEM((CH,), jnp.int32),
        pltpu.VMEM((RBA, D), F32),
        pltpu.VMEM((1, 16), F32),
    ],
)
def _attr_kernel(ln_h, nd_h, attr_h, m_h, a16_out,
                 a16_sh, dstb, lb, wb, attrb, attr_sc, dstch, zb, m_v):
    cid = lax.axis_index("c")
    sid = lax.axis_index("s")
    wid = sid * NC + cid
    base = wid * EPW
    zeros16 = jnp.zeros((16,), F32)

    def zrow(i, c):
        zb[i // 8, pl.ds((i % 8) * 16, 16)] = zeros16
        return c

    lax.fori_loop(0, RBA * 8, zrow, 0)

    def zrow2(i, c):
        attr_sc[i // 8, pl.ds((i % 8) * 16, 16)] = zeros16
        return c

    lax.fori_loop(0, CH * 8, zrow2, 0)

    pltpu.sync_copy(m_h, m_v)
    m16 = m_v[0, :]

    for p in range(2):
        lo = p * HA

        for t in range(8):
            b = sid + t * NS

            @pl.when(b < NZA)
            def _():
                pltpu.sync_copy(zb, a16_sh.at[pl.ds(b * RBA, RBA)])

        plsc.subcore_barrier()

        def group(g, c):
            gb = base + g * GRP_C
            pltpu.sync_copy(nd_h.at[pl.ds(gb, GRP_C)], dstb)
            pltpu.sync_copy(ln_h.at[pl.ds(gb, GRP_C)], lb)
            pltpu.sync_copy(attr_h.at[pl.ds(gb * D_EDGE, GRP_C * D_EDGE)],
                            attrb)

            def wcomp(t, c2):
                wb[pl.ds(t * 16, 16)] = jnp.exp(lb[pl.ds(t * 16, 16)] - m16)
                return c2

            lax.fori_loop(0, GRP_C // 16, wcomp, 0)

            def chunk(ck, c2):
                c0 = ck * CH

                def cpidx(t, c3):
                    d16 = dstb[pl.ds(c0 + t * 16, 16)] - lo
                    ok = jnp.logical_and(d16 >= 0, d16 < HA)
                    dstch[pl.ds(t * 16, 16)] = jnp.where(ok, d16, JROW)
                    return c3

                lax.fori_loop(0, CH // 16, cpidx, 0)

                def scale(e, c3):
                    wspl = plsc.load_gather(
                        wb, [jnp.full((16,), c0 + e, jnp.int32)])
                    attr_sc[e, pl.ds(0, D_EDGE)] = (
                        attrb[pl.ds((c0 + e) * D_EDGE, D_EDGE)] * wspl)
                    return c3

                lax.fori_loop(0, CH, scale, 0)
                pltpu.sync_copy(attr_sc, a16_sh.at[dstch], add=True)
                return c2

            lax.fori_loop(0, NCH, chunk, 0)
            return c

        lax.fori_loop(0, NG_C, group, 0)
        plsc.subcore_barrier()

        for t in range(8):
            b = sid + t * NS

            @pl.when(b < NBA)
            def _():
                pltpu.sync_copy(
                    a16_sh.at[pl.ds(b * RBA, RBA)],
                    a16_out.at[pl.ds(cid * N_RES + lo + b * RBA, RBA)])

        plsc.subcore_barrier()


def kernel(resources, operations, need_edge_attr, need_edge_index,
           same_edge_index, W_self, W_res, W_op, att_self, att_op, att_res):
    ns = need_edge_index[0].astype(jnp.int32)
    nd = need_edge_index[1].astype(jnp.int32)
    ss = same_edge_index[0].astype(jnp.int32)
    sd = same_edge_index[1].astype(jnp.int32)

    a_mat = jnp.concatenate(
        [att_op[:D], att_res[:D], jnp.zeros((D, 1), F32),
         att_self[:D] + att_self[D:]], axis=1)
    b_mat = jnp.concatenate(
        [jnp.zeros((D, 2), F32), att_res[D:], jnp.zeros((D, 1), F32)], axis=1)
    attr_flat = need_edge_attr.reshape(-1)
    attr2 = jnp.pad(attr_flat, (0, 60 * 128 * D_EDGE)).reshape(2560,
                                                              128 * D_EDGE)

    op_proj, s2col = pl.pallas_call(
        _t1_body,
        grid=(N_OPS // BLK1,),
        in_specs=[pl.BlockSpec((BLK1, D_OPF), lambda i: (i, 0)),
                  pl.BlockSpec((D_OPF + D_EDGE, D), lambda i: (0, 0)),
                  pl.BlockSpec((2 * D, 1), lambda i: (0, 0))],
        out_specs=[pl.BlockSpec((BLK1, D), lambda i: (i, 0)),
                   pl.BlockSpec((BLK1, 1), lambda i: (i, 0))],
        out_shape=[jax.ShapeDtypeStruct((N_OPS, D), F32),
                   jax.ShapeDtypeStruct((N_OPS, 1), F32)],
    )(operations, W_op, att_op)

    sr, rp, scal, v2 = pl.pallas_call(
        _t2_body,
        grid=(N_RES // BLK2,),
        in_specs=[pl.BlockSpec((BLK2, D), lambda i: (i, 0)),
                  pl.BlockSpec((D, D), lambda i: (0, 0)),
                  pl.BlockSpec((D, D), lambda i: (0, 0)),
                  pl.BlockSpec((D, 4), lambda i: (0, 0)),
                  pl.BlockSpec((D, 4), lambda i: (0, 0)),
                  pl.BlockSpec((D_OPF + D_EDGE, D), lambda i: (0, 0)),
                  pl.BlockSpec((2 * D, 1), lambda i: (0, 0))],
        out_specs=[pl.BlockSpec((BLK2, D), lambda i: (i, 0)),
                   pl.BlockSpec((BLK2, D), lambda i: (i, 0)),
                   pl.BlockSpec((BLK2, 4), lambda i: (i, 0)),
                   pl.BlockSpec((D_EDGE, 1), lambda i: (0, 0))],
        out_shape=[jax.ShapeDtypeStruct((N_RES, D), F32),
                   jax.ShapeDtypeStruct((N_RES, D), F32),
                   jax.ShapeDtypeStruct((N_RES, 4), F32),
                   jax.ShapeDtypeStruct((D_EDGE, 1), F32)],
    )(resources, W_self, W_res, a_mat, b_mat, W_op, att_op)

    s2f = s2col.reshape(-1)
    scal_flat = scal.reshape(-1)

    vkron = jnp.kron(jnp.eye(128, dtype=F32), v2)
    c2d = pl.pallas_call(
        _t3_body,
        grid=(5,),
        in_specs=[pl.BlockSpec((512, 128 * D_EDGE), lambda i: (i, 0)),
                  pl.BlockSpec((128 * D_EDGE, 128), lambda i: (0, 0))],
        out_specs=pl.BlockSpec((512, 128), lambda i: (i, 0)),
        out_shape=jax.ShapeDtypeStruct((2560, 128), F32),
    )(attr2, vkron)
    c_flat = c2d.reshape(-1)

    lneed, lsame = _logits_kernel(scal_flat, s2f, c_flat, ns, nd, ss, sd)

    m = pl.pallas_call(
        _max_body,
        out_shape=jax.ShapeDtypeStruct((1, 16), F32),
    )(lneed.reshape(E // 128, 128), lsame.reshape(E // 128, 128), scal)

    accp, zpf = _heavy_kernel(op_proj, rp, lneed, lsame, ns, nd, ss, sd, m)
    a16p = _attr_kernel(lneed, nd, attr_flat, m)
    zp = zpf.reshape(NW, 16)

    emb = pl.pallas_call(
        _fin_body,
        grid=(N_RES // BLK2,),
        in_specs=[pl.BlockSpec((BLK2, D), lambda i: (i, 0)),
                  pl.BlockSpec((BLK2, 4), lambda i: (i, 0)),
                  pl.BlockSpec((BLK2, D), lambda i: (i, 0)),
                  pl.BlockSpec((BLK2, D), lambda i: (i + 10, 0)),
                  pl.BlockSpec((BLK2, D), lambda i: (i, 0)),
                  pl.BlockSpec((BLK2, D), lambda i: (i + 10, 0)),
                  pl.BlockSpec((D_OPF + D_EDGE, D), lambda i: (0, 0)),
                  pl.BlockSpec((1, 16), lambda i: (0, 0)),
                  pl.BlockSpec((NW, 16), lambda i: (0, 0)),
                  pl.BlockSpec((N_RES, 4), lambda i: (0, 0))],
        out_specs=pl.BlockSpec((BLK2, D), lambda i: (i, 0)),
        out_shape=jax.ShapeDtypeStruct((N_RES, D), F32),
    )(sr, scal, accp, accp, a16p, a16p, W_op, m, zp, scal)

    return emb


# speedup vs baseline: 5.9386x; 1.0447x over previous
import functools

import jax
import jax.numpy as jnp
from jax import lax
from jax.experimental import pallas as pl
from jax.experimental.pallas import tpu as pltpu
from jax.experimental.pallas import tpu_sc as plsc

N_RES = 10000
N_OPS = 50000
E = 320000
D = 128
D_OPF = 112
D_EDGE = 16
F32 = jnp.float32

NC, NS = 2, 16
NW = NC * NS
EPW = E // NW
GRP_A = 2000
NG_A = EPW // GRP_A
GRP_C = 2000
NG_C = EPW // GRP_C
CH = 80
NCH = GRP_C // CH
RB = 80
NRB = N_RES // RB

BLK1 = 1000
BLK2 = 1000


def _t1_body(ops_ref, wop_ref, attop_ref, op_out, s2_out):
    w1 = wop_ref[:D_OPF, :]
    o = jnp.dot(ops_ref[...], w1, preferred_element_type=F32)
    op_out[...] = o
    s2_out[...] = jnp.dot(o, attop_ref[D:, :], preferred_element_type=F32)


def _t2_body(res_ref, wself_ref, wres_ref, amat_ref, bmat_ref, wop_ref,
             attop_ref, sr_out, rp_out, scal_out, v_out):
    r = res_ref[...]
    sr = jnp.dot(r, wself_ref[...], preferred_element_type=F32)
    rp = jnp.dot(r, wres_ref[...], preferred_element_type=F32)
    sr_out[...] = sr
    rp_out[...] = rp
    scal_out[...] = (jnp.dot(sr, amat_ref[...], preferred_element_type=F32)
                     + jnp.dot(rp, bmat_ref[...], preferred_element_type=F32))
    v_out[...] = jnp.dot(wop_ref[D_OPF:, :], attop_ref[D:, :],
                         preferred_element_type=F32)


def _t3_body(attr2_ref, vkron_ref, c_out):
    c_out[...] = jnp.dot(attr2_ref[...], vkron_ref[...],
                         preferred_element_type=F32)


def _max_body(ln_ref, ls_ref, scal_ref, m_out):
    a = jnp.max(ln_ref[...])
    b = jnp.max(ls_ref[...])
    s3 = scal_ref[:, 3:4]
    c = jnp.max(jnp.where(s3 >= 0, s3, 0.2 * s3))
    bm = jnp.maximum(jnp.maximum(a, b), c)
    m_out[...] = jnp.full((1, 16), bm, F32)


def _fin_body(sr_ref, scal_ref, acc0_ref, acc1_ref, a0_ref, a1_ref, wop_ref,
              m_ref, zp_ref, scalfull_ref, out_ref):
    m = m_ref[0, 0]
    sf = scalfull_ref[:, 3:4]
    lf = jnp.where(sf >= 0, sf, 0.2 * sf)
    z = jnp.sum(jnp.exp(lf - m)) + jnp.sum(zp_ref[...])
    s3 = scal_ref[:, 3:4]
    l3 = jnp.where(s3 >= 0, s3, 0.2 * s3)
    wself = jnp.exp(l3 - m)
    a16 = a0_ref[:, :D_EDGE] + a1_ref[:, :D_EDGE]
    w2 = wop_ref[D_OPF:, :]
    pre = (wself * sr_ref[...] + acc0_ref[...] + acc1_ref[...]
           + jnp.dot(a16, w2, preferred_element_type=F32))
    x = pre / z
    out_ref[...] = jnp.where(x > 0, x, jnp.exp(x) - 1.0)


def _sc_mesh():
    return plsc.VectorSubcoreMesh(core_axis_name="c", subcore_axis_name="s",
                                  num_cores=NC, num_subcores=NS)


@functools.partial(
    pl.kernel,
    out_type=[jax.ShapeDtypeStruct((E,), F32),
              jax.ShapeDtypeStruct((E,), F32)],
    mesh=_sc_mesh(),
    compiler_params=pltpu.CompilerParams(needs_layout_passes=False),
    scratch_types=[
        pltpu.VMEM((4 * N_RES,), F32),
        pltpu.VMEM((N_OPS,), F32),
        pltpu.VMEM((GRP_A,), jnp.int32),
        pltpu.VMEM((GRP_A,), jnp.int32),
        pltpu.VMEM((GRP_A,), F32),
        pltpu.VMEM((GRP_A,), F32),
    ],
)
def _logits_kernel(scal_h, s2_h, c_h, ns_h, nd_h, ss_h, sd_h,
                   lneed_h, lsame_h,
                   scal_v, s2_v, src_v, dst_v, c_v, l_v):
    cid = lax.axis_index("c")
    sid = lax.axis_index("s")
    wid = sid * NC + cid
    base = wid * EPW
    pltpu.sync_copy(scal_h, scal_v)
    pltpu.sync_copy(s2_h, s2_v)

    def need_group(g, carry):
        gb = base + g * GRP_A
        pltpu.sync_copy(ns_h.at[pl.ds(gb, GRP_A)], src_v)
        pltpu.sync_copy(nd_h.at[pl.ds(gb, GRP_A)], dst_v)
        pltpu.sync_copy(c_h.at[pl.ds(gb, GRP_A)], c_v)

        def tile16(t, carry2):
            s16 = src_v[pl.ds(t * 16, 16)]
            d16 = dst_v[pl.ds(t * 16, 16)]
            g1 = plsc.load_gather(scal_v, [d16 * 4])
            g2 = plsc.load_gather(s2_v, [s16])
            x = g1 + g2 + c_v[pl.ds(t * 16, 16)]
            l_v[pl.ds(t * 16, 16)] = jnp.where(x >= 0, x, 0.2 * x)
            return carry2

        lax.fori_loop(0, GRP_A // 16, tile16, 0)
        pltpu.sync_copy(l_v, lneed_h.at[pl.ds(gb, GRP_A)])
        return carry

    lax.fori_loop(0, NG_A, need_group, 0)

    def same_group(g, carry):
        gb = base + g * GRP_A
        pltpu.sync_copy(ss_h.at[pl.ds(gb, GRP_A)], src_v)
        pltpu.sync_copy(sd_h.at[pl.ds(gb, GRP_A)], dst_v)

        def tile16(t, carry2):
            s16 = src_v[pl.ds(t * 16, 16)]
            d16 = dst_v[pl.ds(t * 16, 16)]
            g1 = plsc.load_gather(scal_v, [d16 * 4 + 1])
            g2 = plsc.load_gather(scal_v, [s16 * 4 + 2])
            x = g1 + g2
            l_v[pl.ds(t * 16, 16)] = jnp.where(x >= 0, x, 0.2 * x)
            return carry2

        lax.fori_loop(0, GRP_A // 16, tile16, 0)
        pltpu.sync_copy(l_v, lsame_h.at[pl.ds(gb, GRP_A)])
        return carry

    lax.fori_loop(0, NG_A, same_group, 0)


@functools.partial(
    pl.kernel,
    out_type=[jax.ShapeDtypeStruct((2 * N_RES, D), F32),
              jax.ShapeDtypeStruct((NW * 16,), F32)],
    mesh=_sc_mesh(),
    compiler_params=pltpu.CompilerParams(needs_layout_passes=False),
    scratch_types=[
        pltpu.VMEM_SHARED((N_RES, D), F32),
        pltpu.VMEM((GRP_C,), jnp.int32),
        pltpu.VMEM((GRP_C,), jnp.int32),
        pltpu.VMEM((GRP_C,), F32),
        pltpu.VMEM((GRP_C,), F32),
        pltpu.VMEM((2, CH, D), F32),
        pltpu.VMEM((2, CH), jnp.int32),
        pltpu.VMEM((2, CH), jnp.int32),
        pltpu.VMEM((RB, D), F32),
        pltpu.VMEM((1, 16), F32),
        pltpu.VMEM((16,), F32),
        pltpu.SemaphoreType.DMA,
    ],
)
def _heavy_kernel(op_h, rp_h, ln_h, ls_h, ns_h, nd_h, ss_h, sd_h, m_h,
                  acc_out, z_out,
                  acc_sh, srcb, dstb, lb, wb, rows,
                  srcch, dstch, zb, m_v, zsum_v, sem):
    cid = lax.axis_index("c")
    sid = lax.axis_index("s")
    wid = sid * NC + cid
    base = wid * EPW
    zeros16 = jnp.zeros((16,), F32)

    def zrow(i, c):
        zb[i // 8, pl.ds((i % 8) * 16, 16)] = zeros16
        return c

    lax.fori_loop(0, RB * 8, zrow, 0)

    for t in range(8):
        b = sid + t * NS

        @pl.when(b < NRB)
        def _():
            pltpu.sync_copy(zb, acc_sh.at[pl.ds(b * RB, RB)])

    pltpu.sync_copy(m_h, m_v)
    plsc.subcore_barrier()

    m16 = m_v[0, :]

    def run_phase(table_h, s_h, d_h, l_h, zacc0):
        def group(g, zacc):
            gb = base + g * GRP_C
            pltpu.sync_copy(s_h.at[pl.ds(gb, GRP_C)], srcb)
            pltpu.sync_copy(d_h.at[pl.ds(gb, GRP_C)], dstb)
            pltpu.sync_copy(l_h.at[pl.ds(gb, GRP_C)], lb)

            def wcomp(t, za_):
                w16 = jnp.exp(lb[pl.ds(t * 16, 16)] - m16)
                wb[pl.ds(t * 16, 16)] = w16
                return za_ + w16

            zacc = lax.fori_loop(0, GRP_C // 16, wcomp, zacc)

            def build(ck, b):
                c0 = ck * CH

                def cpidx(t, c2):
                    srcch[b, pl.ds(t * 16, 16)] = srcb[pl.ds(c0 + t * 16, 16)]
                    dstch[b, pl.ds(t * 16, 16)] = dstb[pl.ds(c0 + t * 16, 16)]
                    return c2

                lax.fori_loop(0, CH // 16, cpidx, 0)

            def start(b):
                pltpu.async_copy(table_h.at[srcch.at[b]], rows.at[b], sem)

            def drain(b):
                pltpu.make_async_copy(table_h.at[srcch.at[b]], rows.at[b],
                                      sem).wait()

            def consume(ck, b):
                def scale(j, c2):
                    e = 2 * j
                    w0 = plsc.load_gather(
                        wb, [jnp.full((16,), ck * CH + e, jnp.int32)])
                    w1 = plsc.load_gather(
                        wb, [jnp.full((16,), ck * CH + e + 1, jnp.int32)])
                    for k in range(D // 16):
                        rows[b, e, pl.ds(k * 16, 16)] = (
                            rows[b, e, pl.ds(k * 16, 16)] * w0)
                    for k in range(D // 16):
                        rows[b, e + 1, pl.ds(k * 16, 16)] = (
                            rows[b, e + 1, pl.ds(k * 16, 16)] * w1)
                    return c2

                lax.fori_loop(0, CH // 2, scale, 0)
                pltpu.sync_copy(rows.at[b], acc_sh.at[dstch.at[b]], add=True)

            build(0, 0)
            start(0)

            def pair(i, c):
                ck0 = 2 * i
                build(ck0 + 1, 1)
                start(1)
                drain(0)
                consume(ck0, 0)
                build(ck0 + 2, 0)
                start(0)
                drain(1)
                consume(ck0 + 1, 1)
                return c

            lax.fori_loop(0, NCH // 2, pair, 0)
            drain(0)
            consume(NCH - 1, 0)
            return zacc

        return lax.fori_loop(0, NG_C, group, zacc0)

    zacc = jnp.zeros((16,), F32)
    zacc = run_phase(op_h, ns_h, nd_h, ln_h, zacc)
    zacc = run_phase(rp_h, ss_h, sd_h, ls_h, zacc)
    zsum_v[...] = zacc
    pltpu.sync_copy(zsum_v, z_out.at[pl.ds(wid * 16, 16)])
    plsc.subcore_barrier()

    for t in range(8):
        b = sid + t * NS

        @pl.when(b < NRB)
        def _():
            pltpu.sync_copy(acc_sh.at[pl.ds(b * RB, RB)],
                            acc_out.at[pl.ds(cid * N_RES + b * RB, RB)])


HA = N_RES // 2
JROW = HA
RBA = 40
NBA = HA // RBA
NZA = NBA + 1


@functools.partial(
    pl.kernel,
    out_type=jax.ShapeDtypeStruct((2 * N_RES, D), F32),
    mesh=_sc_mesh(),
    compiler_params=pltpu.CompilerParams(needs_layout_passes=False),
    scratch_types=[
        pltpu.VMEM_SHARED((HA + RBA, D), F32),
        pltpu.VMEM((GRP_C,), jnp.int32),
        pltpu.VMEM((GRP_C,), F32),
        pltpu.VMEM((GRP_C,), F32),
        pltpu.VMEM((GRP_C * D_EDGE,), F32),
        pltpu.VMEM((2, CH, D), F32),
        pltpu.VMEM((2, CH), jnp.int32),
        pltpu.VMEM((RBA, D), F32),
        pltpu.VMEM((1, 16), F32),
        pltpu.SemaphoreType.DMA,
    ],
)
def _attr_kernel(ln_h, nd_h, attr_h, m_h, a16_out,
                 a16_sh, dstb, lb, wb, attrb, attr_sc, dstch, zb, m_v, ssem):
    cid = lax.axis_index("c")
    sid = lax.axis_index("s")
    wid = sid * NC + cid
    base = wid * EPW
    zeros16 = jnp.zeros((16,), F32)

    def zrow(i, c):
        zb[i // 8, pl.ds((i % 8) * 16, 16)] = zeros16
        return c

    lax.fori_loop(0, RBA * 8, zrow, 0)

    def zrow2(i, c):
        attr_sc[i // (CH * 8), (i // 8) % CH, pl.ds((i % 8) * 16, 16)] = (
            zeros16)
        return c

    lax.fori_loop(0, 2 * CH * 8, zrow2, 0)

    pltpu.sync_copy(m_h, m_v)
    m16 = m_v[0, :]

    for p in range(2):
        lo = p * HA

        for t in range(8):
            b = sid + t * NS

            @pl.when(b < NZA)
            def _():
                pltpu.sync_copy(zb, a16_sh.at[pl.ds(b * RBA, RBA)])

        plsc.subcore_barrier()

        def group(g, c):
            gb = base + g * GRP_C
            pltpu.sync_copy(nd_h.at[pl.ds(gb, GRP_C)], dstb)
            pltpu.sync_copy(ln_h.at[pl.ds(gb, GRP_C)], lb)
            pltpu.sync_copy(attr_h.at[pl.ds(gb * D_EDGE, GRP_C * D_EDGE)],
                            attrb)

            def wcomp(t, c2):
                wb[pl.ds(t * 16, 16)] = jnp.exp(lb[pl.ds(t * 16, 16)] - m16)
                return c2

            lax.fori_loop(0, GRP_C // 16, wcomp, 0)

            def prep(ck, b):
                c0 = ck * CH

                def cpidx(t, c3):
                    d16 = dstb[pl.ds(c0 + t * 16, 16)] - lo
                    ok = jnp.logical_and(d16 >= 0, d16 < HA)
                    dstch[b, pl.ds(t * 16, 16)] = jnp.where(ok, d16, JROW)
                    return c3

                lax.fori_loop(0, CH // 16, cpidx, 0)

                def scale(j, c3):
                    e = 2 * j
                    w0 = plsc.load_gather(
                        wb, [jnp.full((16,), c0 + e, jnp.int32)])
                    w1 = plsc.load_gather(
                        wb, [jnp.full((16,), c0 + e + 1, jnp.int32)])
                    attr_sc[b, e, pl.ds(0, D_EDGE)] = (
                        attrb[pl.ds((c0 + e) * D_EDGE, D_EDGE)] * w0)
                    attr_sc[b, e + 1, pl.ds(0, D_EDGE)] = (
                        attrb[pl.ds((c0 + e + 1) * D_EDGE, D_EDGE)] * w1)
                    return c3

                lax.fori_loop(0, CH // 2, scale, 0)

            def sc_start(b):
                pltpu.async_copy(attr_sc.at[b], a16_sh.at[dstch.at[b]], ssem,
                                 add=True)

            def sc_drain(b):
                pltpu.make_async_copy(attr_sc.at[b],
                                      a16_sh.at[dstch.at[b]], ssem).wait()

            prep(0, 0)
            sc_start(0)

            def pair(i, c2):
                ck0 = 2 * i
                prep(ck0 + 1, 1)
                sc_start(1)
                sc_drain(0)
                prep(ck0 + 2, 0)
                sc_start(0)
                sc_drain(1)
                return c2

            lax.fori_loop(0, NCH // 2, pair, 0)
            sc_drain(0)
            return c

        lax.fori_loop(0, NG_C, group, 0)
        plsc.subcore_barrier()

        for t in range(8):
            b = sid + t * NS

            @pl.when(b < NBA)
            def _():
                pltpu.sync_copy(
                    a16_sh.at[pl.ds(b * RBA, RBA)],
                    a16_out.at[pl.ds(cid * N_RES + lo + b * RBA, RBA)])

        plsc.subcore_barrier()


def kernel(resources, operations, need_edge_attr, need_edge_index,
           same_edge_index, W_self, W_res, W_op, att_self, att_op, att_res):
    ns = need_edge_index[0].astype(jnp.int32)
    nd = need_edge_index[1].astype(jnp.int32)
    ss = same_edge_index[0].astype(jnp.int32)
    sd = same_edge_index[1].astype(jnp.int32)

    a_mat = jnp.concatenate(
        [att_op[:D], att_res[:D], jnp.zeros((D, 1), F32),
         att_self[:D] + att_self[D:]], axis=1)
    b_mat = jnp.concatenate(
        [jnp.zeros((D, 2), F32), att_res[D:], jnp.zeros((D, 1), F32)], axis=1)
    attr_flat = need_edge_attr.reshape(-1)
    attr2 = jnp.pad(attr_flat, (0, 60 * 128 * D_EDGE)).reshape(2560,
                                                              128 * D_EDGE)

    op_proj, s2col = pl.pallas_call(
        _t1_body,
        grid=(N_OPS // BLK1,),
        in_specs=[pl.BlockSpec((BLK1, D_OPF), lambda i: (i, 0)),
                  pl.BlockSpec((D_OPF + D_EDGE, D), lambda i: (0, 0)),
                  pl.BlockSpec((2 * D, 1), lambda i: (0, 0))],
        out_specs=[pl.BlockSpec((BLK1, D), lambda i: (i, 0)),
                   pl.BlockSpec((BLK1, 1), lambda i: (i, 0))],
        out_shape=[jax.ShapeDtypeStruct((N_OPS, D), F32),
                   jax.ShapeDtypeStruct((N_OPS, 1), F32)],
    )(operations, W_op, att_op)

    sr, rp, scal, v2 = pl.pallas_call(
        _t2_body,
        grid=(N_RES // BLK2,),
        in_specs=[pl.BlockSpec((BLK2, D), lambda i: (i, 0)),
                  pl.BlockSpec((D, D), lambda i: (0, 0)),
                  pl.BlockSpec((D, D), lambda i: (0, 0)),
                  pl.BlockSpec((D, 4), lambda i: (0, 0)),
                  pl.BlockSpec((D, 4), lambda i: (0, 0)),
                  pl.BlockSpec((D_OPF + D_EDGE, D), lambda i: (0, 0)),
                  pl.BlockSpec((2 * D, 1), lambda i: (0, 0))],
        out_specs=[pl.BlockSpec((BLK2, D), lambda i: (i, 0)),
                   pl.BlockSpec((BLK2, D), lambda i: (i, 0)),
                   pl.BlockSpec((BLK2, 4), lambda i: (i, 0)),
                   pl.BlockSpec((D_EDGE, 1), lambda i: (0, 0))],
        out_shape=[jax.ShapeDtypeStruct((N_RES, D), F32),
                   jax.ShapeDtypeStruct((N_RES, D), F32),
                   jax.ShapeDtypeStruct((N_RES, 4), F32),
                   jax.ShapeDtypeStruct((D_EDGE, 1), F32)],
    )(resources, W_self, W_res, a_mat, b_mat, W_op, att_op)

    s2f = s2col.reshape(-1)
    scal_flat = scal.reshape(-1)

    vkron = jnp.kron(jnp.eye(128, dtype=F32), v2)
    c2d = pl.pallas_call(
        _t3_body,
        grid=(5,),
        in_specs=[pl.BlockSpec((512, 128 * D_EDGE), lambda i: (i, 0)),
                  pl.BlockSpec((128 * D_EDGE, 128), lambda i: (0, 0))],
        out_specs=pl.BlockSpec((512, 128), lambda i: (i, 0)),
        out_shape=jax.ShapeDtypeStruct((2560, 128), F32),
    )(attr2, vkron)
    c_flat = c2d.reshape(-1)

    lneed, lsame = _logits_kernel(scal_flat, s2f, c_flat, ns, nd, ss, sd)

    m = pl.pallas_call(
        _max_body,
        out_shape=jax.ShapeDtypeStruct((1, 16), F32),
    )(lneed.reshape(E // 128, 128), lsame.reshape(E // 128, 128), scal)

    accp, zpf = _heavy_kernel(op_proj, rp, lneed, lsame, ns, nd, ss, sd, m)
    a16p = _attr_kernel(lneed, nd, attr_flat, m)
    zp = zpf.reshape(NW, 16)

    emb = pl.pallas_call(
        _fin_body,
        grid=(N_RES // BLK2,),
        in_specs=[pl.BlockSpec((BLK2, D), lambda i: (i, 0)),
                  pl.BlockSpec((BLK2, 4), lambda i: (i, 0)),
                  pl.BlockSpec((BLK2, D), lambda i: (i, 0)),
                  pl.BlockSpec((BLK2, D), lambda i: (i + 10, 0)),
                  pl.BlockSpec((BLK2, D), lambda i: (i, 0)),
                  pl.BlockSpec((BLK2, D), lambda i: (i + 10, 0)),
                  pl.BlockSpec((D_OPF + D_EDGE, D), lambda i: (0, 0)),
                  pl.BlockSpec((1, 16), lambda i: (0, 0)),
                  pl.BlockSpec((NW, 16), lambda i: (0, 0)),
                  pl.BlockSpec((N_RES, 4), lambda i: (0, 0))],
        out_specs=pl.BlockSpec((BLK2, D), lambda i: (i, 0)),
        out_shape=jax.ShapeDtypeStruct((N_RES, D), F32),
    )(sr, scal, accp, accp, a16p, a16p, W_op, m, zp, scal)

    return emb


# speedup vs baseline: 6.2081x; 1.0454x over previous
import functools

import jax
import jax.numpy as jnp
from jax import lax
from jax.experimental import pallas as pl
from jax.experimental.pallas import tpu as pltpu
from jax.experimental.pallas import tpu_sc as plsc

N_RES = 10000
N_OPS = 50000
E = 320000
D = 128
D_OPF = 112
D_EDGE = 16
F32 = jnp.float32

NC, NS = 2, 16
NW = NC * NS
EPW = E // NW
GRP_A = 2000
NG_A = EPW // GRP_A
GRP_C = 2000
NG_C = EPW // GRP_C
CH = 80
NCH = GRP_C // CH
RB = 80
NRB = N_RES // RB

BLK1 = 1000
BLK2 = 1000


def _t1_body(ops_ref, wop_ref, attop_ref, op_out, s2_out):
    w1 = wop_ref[:D_OPF, :]
    o = jnp.dot(ops_ref[...], w1, preferred_element_type=F32)
    op_out[...] = o
    s2_out[...] = jnp.dot(o, attop_ref[D:, :], preferred_element_type=F32)


def _t2_body(res_ref, wself_ref, wres_ref, amat_ref, bmat_ref, wop_ref,
             attop_ref, sr_out, rp_out, scal_out, v_out):
    r = res_ref[...]
    sr = jnp.dot(r, wself_ref[...], preferred_element_type=F32)
    rp = jnp.dot(r, wres_ref[...], preferred_element_type=F32)
    sr_out[...] = sr
    rp_out[...] = rp
    scal_out[...] = (jnp.dot(sr, amat_ref[...], preferred_element_type=F32)
                     + jnp.dot(rp, bmat_ref[...], preferred_element_type=F32))
    v_out[...] = jnp.dot(wop_ref[D_OPF:, :], attop_ref[D:, :],
                         preferred_element_type=F32)


def _t3_body(attr2_ref, vkron_ref, c_out):
    c_out[...] = jnp.dot(attr2_ref[...], vkron_ref[...],
                         preferred_element_type=F32)


def _max_body(ln_ref, ls_ref, scal_ref, m_out):
    a = jnp.max(ln_ref[...])
    b = jnp.max(ls_ref[...])
    s3 = scal_ref[:, 3:4]
    c = jnp.max(jnp.where(s3 >= 0, s3, 0.2 * s3))
    bm = jnp.maximum(jnp.maximum(a, b), c)
    m_out[...] = jnp.full((1, 16), bm, F32)


def _fin_body(sr_ref, scal_ref, acc0_ref, acc1_ref, a0_ref, a1_ref, wop_ref,
              m_ref, zp_ref, scalfull_ref, out_ref):
    m = m_ref[0, 0]
    sf = scalfull_ref[:, 3:4]
    lf = jnp.where(sf >= 0, sf, 0.2 * sf)
    z = jnp.sum(jnp.exp(lf - m)) + jnp.sum(zp_ref[...])
    s3 = scal_ref[:, 3:4]
    l3 = jnp.where(s3 >= 0, s3, 0.2 * s3)
    wself = jnp.exp(l3 - m)
    a16 = a0_ref[:, :D_EDGE] + a1_ref[:, :D_EDGE]
    w2 = wop_ref[D_OPF:, :]
    pre = (wself * sr_ref[...] + acc0_ref[...] + acc1_ref[...]
           + jnp.dot(a16, w2, preferred_element_type=F32))
    x = pre / z
    out_ref[...] = jnp.where(x > 0, x, jnp.exp(x) - 1.0)


def _sc_mesh():
    return plsc.VectorSubcoreMesh(core_axis_name="c", subcore_axis_name="s",
                                  num_cores=NC, num_subcores=NS)


@functools.partial(
    pl.kernel,
    out_type=[jax.ShapeDtypeStruct((E,), F32),
              jax.ShapeDtypeStruct((E,), F32)],
    mesh=_sc_mesh(),
    compiler_params=pltpu.CompilerParams(needs_layout_passes=False),
    scratch_types=[
        pltpu.VMEM((4 * N_RES,), F32),
        pltpu.VMEM((N_OPS,), F32),
        pltpu.VMEM((GRP_A,), jnp.int32),
        pltpu.VMEM((GRP_A,), jnp.int32),
        pltpu.VMEM((GRP_A,), F32),
        pltpu.VMEM((GRP_A,), F32),
    ],
)
def _logits_kernel(scal_h, s2_h, c_h, ns_h, nd_h, ss_h, sd_h,
                   lneed_h, lsame_h,
                   scal_v, s2_v, src_v, dst_v, c_v, l_v):
    cid = lax.axis_index("c")
    sid = lax.axis_index("s")
    wid = sid * NC + cid
    base = wid * EPW
    pltpu.sync_copy(scal_h, scal_v)
    pltpu.sync_copy(s2_h, s2_v)

    def need_group(g, carry):
        gb = base + g * GRP_A
        pltpu.sync_copy(ns_h.at[pl.ds(gb, GRP_A)], src_v)
        pltpu.sync_copy(nd_h.at[pl.ds(gb, GRP_A)], dst_v)
        pltpu.sync_copy(c_h.at[pl.ds(gb, GRP_A)], c_v)

        def tile16(t, carry2):
            s16 = src_v[pl.ds(t * 16, 16)]
            d16 = dst_v[pl.ds(t * 16, 16)]
            g1 = plsc.load_gather(scal_v, [d16 * 4])
            g2 = plsc.load_gather(s2_v, [s16])
            x = g1 + g2 + c_v[pl.ds(t * 16, 16)]
            l_v[pl.ds(t * 16, 16)] = jnp.where(x >= 0, x, 0.2 * x)
            return carry2

        lax.fori_loop(0, GRP_A // 16, tile16, 0)
        pltpu.sync_copy(l_v, lneed_h.at[pl.ds(gb, GRP_A)])
        return carry

    lax.fori_loop(0, NG_A, need_group, 0)

    def same_group(g, carry):
        gb = base + g * GRP_A
        pltpu.sync_copy(ss_h.at[pl.ds(gb, GRP_A)], src_v)
        pltpu.sync_copy(sd_h.at[pl.ds(gb, GRP_A)], dst_v)

        def tile16(t, carry2):
            s16 = src_v[pl.ds(t * 16, 16)]
            d16 = dst_v[pl.ds(t * 16, 16)]
            g1 = plsc.load_gather(scal_v, [d16 * 4 + 1])
            g2 = plsc.load_gather(scal_v, [s16 * 4 + 2])
            x = g1 + g2
            l_v[pl.ds(t * 16, 16)] = jnp.where(x >= 0, x, 0.2 * x)
            return carry2

        lax.fori_loop(0, GRP_A // 16, tile16, 0)
        pltpu.sync_copy(l_v, lsame_h.at[pl.ds(gb, GRP_A)])
        return carry

    lax.fori_loop(0, NG_A, same_group, 0)


@functools.partial(
    pl.kernel,
    out_type=[jax.ShapeDtypeStruct((2 * N_RES, D), F32),
              jax.ShapeDtypeStruct((NW * 16,), F32)],
    mesh=_sc_mesh(),
    compiler_params=pltpu.CompilerParams(needs_layout_passes=False),
    scratch_types=[
        pltpu.VMEM_SHARED((N_RES, D), F32),
        pltpu.VMEM((GRP_C,), jnp.int32),
        pltpu.VMEM((GRP_C,), jnp.int32),
        pltpu.VMEM((GRP_C,), F32),
        pltpu.VMEM((GRP_C,), F32),
        pltpu.VMEM((3, CH, D), F32),
        pltpu.VMEM((3, CH), jnp.int32),
        pltpu.VMEM((3, CH), jnp.int32),
        pltpu.VMEM((RB, D), F32),
        pltpu.VMEM((1, 16), F32),
        pltpu.VMEM((16,), F32),
        pltpu.SemaphoreType.DMA,
        pltpu.SemaphoreType.DMA,
    ],
)
def _heavy_kernel(op_h, rp_h, ln_h, ls_h, ns_h, nd_h, ss_h, sd_h, m_h,
                  acc_out, z_out,
                  acc_sh, srcb, dstb, lb, wb, rows,
                  srcch, dstch, zb, m_v, zsum_v, sem, ssem):
    cid = lax.axis_index("c")
    sid = lax.axis_index("s")
    wid = sid * NC + cid
    base = wid * EPW
    zeros16 = jnp.zeros((16,), F32)

    def zrow(i, c):
        zb[i // 8, pl.ds((i % 8) * 16, 16)] = zeros16
        return c

    lax.fori_loop(0, RB * 8, zrow, 0)

    for t in range(8):
        b = sid + t * NS

        @pl.when(b < NRB)
        def _():
            pltpu.sync_copy(zb, acc_sh.at[pl.ds(b * RB, RB)])

    pltpu.sync_copy(m_h, m_v)
    plsc.subcore_barrier()

    m16 = m_v[0, :]

    def run_phase(table_h, s_h, d_h, l_h, zacc0):
        def group(g, zacc):
            gb = base + g * GRP_C
            pltpu.sync_copy(s_h.at[pl.ds(gb, GRP_C)], srcb)
            pltpu.sync_copy(d_h.at[pl.ds(gb, GRP_C)], dstb)
            pltpu.sync_copy(l_h.at[pl.ds(gb, GRP_C)], lb)

            def wcomp(t, za_):
                w16 = jnp.exp(lb[pl.ds(t * 16, 16)] - m16)
                wb[pl.ds(t * 16, 16)] = w16
                return za_ + w16

            zacc = lax.fori_loop(0, GRP_C // 16, wcomp, zacc)

            def build(ck, b):
                c0 = ck * CH

                def cpidx(t, c2):
                    srcch[b, pl.ds(t * 16, 16)] = srcb[pl.ds(c0 + t * 16, 16)]
                    dstch[b, pl.ds(t * 16, 16)] = dstb[pl.ds(c0 + t * 16, 16)]
                    return c2

                lax.fori_loop(0, CH // 16, cpidx, 0)

            def start(b):
                pltpu.async_copy(table_h.at[srcch.at[b]], rows.at[b], sem)

            def drain(b):
                pltpu.make_async_copy(table_h.at[srcch.at[b]], rows.at[b],
                                      sem).wait()

            def scale(ck, b):
                def sc2(j, c2):
                    e = 2 * j
                    w0 = plsc.load_gather(
                        wb, [jnp.full((16,), ck * CH + e, jnp.int32)])
                    w1 = plsc.load_gather(
                        wb, [jnp.full((16,), ck * CH + e + 1, jnp.int32)])
                    for k in range(D // 16):
                        rows[b, e, pl.ds(k * 16, 16)] = (
                            rows[b, e, pl.ds(k * 16, 16)] * w0)
                    for k in range(D // 16):
                        rows[b, e + 1, pl.ds(k * 16, 16)] = (
                            rows[b, e + 1, pl.ds(k * 16, 16)] * w1)
                    return c2

                lax.fori_loop(0, CH // 2, sc2, 0)

            def sc_start(b):
                pltpu.async_copy(rows.at[b], acc_sh.at[dstch.at[b]], ssem,
                                 add=True)

            def sc_drain(b):
                pltpu.make_async_copy(rows.at[b], acc_sh.at[dstch.at[b]],
                                      ssem).wait()

            build(0, 0)
            start(0)
            build(1, 1)
            start(1)
            drain(0)
            scale(0, 0)
            sc_start(0)
            build(2, 2)
            start(2)

            def tri(i, c):
                for o, b in ((1, 1), (2, 2), (3, 0)):
                    ck = 3 * i + o
                    drain(b)
                    scale(ck, b)
                    sc_start(b)
                    p = (b + 2) % 3

                    @pl.when(ck + 2 < NCH)
                    def _():
                        sc_drain(p)
                        build(ck + 2, p)
                        start(p)

                return c

            lax.fori_loop(0, (NCH - 1) // 3, tri, 0)
            sc_drain(1)
            sc_drain(2)
            sc_drain(0)
            return zacc

        return lax.fori_loop(0, NG_C, group, zacc0)

    zacc = jnp.zeros((16,), F32)
    zacc = run_phase(op_h, ns_h, nd_h, ln_h, zacc)
    zacc = run_phase(rp_h, ss_h, sd_h, ls_h, zacc)
    zsum_v[...] = zacc
    pltpu.sync_copy(zsum_v, z_out.at[pl.ds(wid * 16, 16)])
    plsc.subcore_barrier()

    for t in range(8):
        b = sid + t * NS

        @pl.when(b < NRB)
        def _():
            pltpu.sync_copy(acc_sh.at[pl.ds(b * RB, RB)],
                            acc_out.at[pl.ds(cid * N_RES + b * RB, RB)])


HA = N_RES // 2
JROW = HA
RBA = 40
NBA = HA // RBA
NZA = NBA + 1


@functools.partial(
    pl.kernel,
    out_type=jax.ShapeDtypeStruct((2 * N_RES, D), F32),
    mesh=_sc_mesh(),
    compiler_params=pltpu.CompilerParams(needs_layout_passes=False),
    scratch_types=[
        pltpu.VMEM_SHARED((HA + RBA, D), F32),
        pltpu.VMEM((GRP_C,), jnp.int32),
        pltpu.VMEM((GRP_C,), F32),
        pltpu.VMEM((GRP_C,), F32),
        pltpu.VMEM((GRP_C * D_EDGE,), F32),
        pltpu.VMEM((2, CH, D), F32),
        pltpu.VMEM((2, CH), jnp.int32),
        pltpu.VMEM((RBA, D), F32),
        pltpu.VMEM((1, 16), F32),
        pltpu.SemaphoreType.DMA,
    ],
)
def _attr_kernel(ln_h, nd_h, attr_h, m_h, a16_out,
                 a16_sh, dstb, lb, wb, attrb, attr_sc, dstch, zb, m_v, ssem):
    cid = lax.axis_index("c")
    sid = lax.axis_index("s")
    wid = sid * NC + cid
    base = wid * EPW
    zeros16 = jnp.zeros((16,), F32)

    def zrow(i, c):
        zb[i // 8, pl.ds((i % 8) * 16, 16)] = zeros16
        return c

    lax.fori_loop(0, RBA * 8, zrow, 0)

    def zrow2(i, c):
        attr_sc[i // (CH * 8), (i // 8) % CH, pl.ds((i % 8) * 16, 16)] = (
            zeros16)
        return c

    lax.fori_loop(0, 2 * CH * 8, zrow2, 0)

    pltpu.sync_copy(m_h, m_v)
    m16 = m_v[0, :]

    for p in range(2):
        lo = p * HA

        for t in range(8):
            b = sid + t * NS

            @pl.when(b < NZA)
            def _():
                pltpu.sync_copy(zb, a16_sh.at[pl.ds(b * RBA, RBA)])

        plsc.subcore_barrier()

        def group(g, c):
            gb = base + g * GRP_C
            pltpu.sync_copy(nd_h.at[pl.ds(gb, GRP_C)], dstb)
            pltpu.sync_copy(ln_h.at[pl.ds(gb, GRP_C)], lb)
            pltpu.sync_copy(attr_h.at[pl.ds(gb * D_EDGE, GRP_C * D_EDGE)],
                            attrb)

            def wcomp(t, c2):
                wb[pl.ds(t * 16, 16)] = jnp.exp(lb[pl.ds(t * 16, 16)] - m16)
                return c2

            lax.fori_loop(0, GRP_C // 16, wcomp, 0)

            def prep(ck, b):
                c0 = ck * CH

                def cpidx(t, c3):
                    d16 = dstb[pl.ds(c0 + t * 16, 16)] - lo
                    ok = jnp.logical_and(d16 >= 0, d16 < HA)
                    dstch[b, pl.ds(t * 16, 16)] = jnp.where(ok, d16, JROW)
                    return c3

                lax.fori_loop(0, CH // 16, cpidx, 0)

                def scale(j, c3):
                    e = 2 * j
                    w0 = plsc.load_gather(
                        wb, [jnp.full((16,), c0 + e, jnp.int32)])
                    w1 = plsc.load_gather(
                        wb, [jnp.full((16,), c0 + e + 1, jnp.int32)])
                    attr_sc[b, e, pl.ds(0, D_EDGE)] = (
                        attrb[pl.ds((c0 + e) * D_EDGE, D_EDGE)] * w0)
                    attr_sc[b, e + 1, pl.ds(0, D_EDGE)] = (
                        attrb[pl.ds((c0 + e + 1) * D_EDGE, D_EDGE)] * w1)
                    return c3

                lax.fori_loop(0, CH // 2, scale, 0)

            def sc_start(b):
                pltpu.async_copy(attr_sc.at[b], a16_sh.at[dstch.at[b]], ssem,
                                 add=True)

            def sc_drain(b):
                pltpu.make_async_copy(attr_sc.at[b],
                                      a16_sh.at[dstch.at[b]], ssem).wait()

            prep(0, 0)
            sc_start(0)

            def pair(i, c2):
                ck0 = 2 * i
                prep(ck0 + 1, 1)
                sc_start(1)
                sc_drain(0)
                prep(ck0 + 2, 0)
                sc_start(0)
                sc_drain(1)
                return c2

            lax.fori_loop(0, NCH // 2, pair, 0)
            sc_drain(0)
            return c

        lax.fori_loop(0, NG_C, group, 0)
        plsc.subcore_barrier()

        for t in range(8):
            b = sid + t * NS

            @pl.when(b < NBA)
            def _():
                pltpu.sync_copy(
                    a16_sh.at[pl.ds(b * RBA, RBA)],
                    a16_out.at[pl.ds(cid * N_RES + lo + b * RBA, RBA)])

        plsc.subcore_barrier()


def kernel(resources, operations, need_edge_attr, need_edge_index,
           same_edge_index, W_self, W_res, W_op, att_self, att_op, att_res):
    ns = need_edge_index[0].astype(jnp.int32)
    nd = need_edge_index[1].astype(jnp.int32)
    ss = same_edge_index[0].astype(jnp.int32)
    sd = same_edge_index[1].astype(jnp.int32)

    a_mat = jnp.concatenate(
        [att_op[:D], att_res[:D], jnp.zeros((D, 1), F32),
         att_self[:D] + att_self[D:]], axis=1)
    b_mat = jnp.concatenate(
        [jnp.zeros((D, 2), F32), att_res[D:], jnp.zeros((D, 1), F32)], axis=1)
    attr_flat = need_edge_attr.reshape(-1)
    attr2 = jnp.pad(attr_flat, (0, 60 * 128 * D_EDGE)).reshape(2560,
                                                              128 * D_EDGE)

    op_proj, s2col = pl.pallas_call(
        _t1_body,
        grid=(N_OPS // BLK1,),
        in_specs=[pl.BlockSpec((BLK1, D_OPF), lambda i: (i, 0)),
                  pl.BlockSpec((D_OPF + D_EDGE, D), lambda i: (0, 0)),
                  pl.BlockSpec((2 * D, 1), lambda i: (0, 0))],
        out_specs=[pl.BlockSpec((BLK1, D), lambda i: (i, 0)),
                   pl.BlockSpec((BLK1, 1), lambda i: (i, 0))],
        out_shape=[jax.ShapeDtypeStruct((N_OPS, D), F32),
                   jax.ShapeDtypeStruct((N_OPS, 1), F32)],
    )(operations, W_op, att_op)

    sr, rp, scal, v2 = pl.pallas_call(
        _t2_body,
        grid=(N_RES // BLK2,),
        in_specs=[pl.BlockSpec((BLK2, D), lambda i: (i, 0)),
                  pl.BlockSpec((D, D), lambda i: (0, 0)),
                  pl.BlockSpec((D, D), lambda i: (0, 0)),
                  pl.BlockSpec((D, 4), lambda i: (0, 0)),
                  pl.BlockSpec((D, 4), lambda i: (0, 0)),
                  pl.BlockSpec((D_OPF + D_EDGE, D), lambda i: (0, 0)),
                  pl.BlockSpec((2 * D, 1), lambda i: (0, 0))],
        out_specs=[pl.BlockSpec((BLK2, D), lambda i: (i, 0)),
                   pl.BlockSpec((BLK2, D), lambda i: (i, 0)),
                   pl.BlockSpec((BLK2, 4), lambda i: (i, 0)),
                   pl.BlockSpec((D_EDGE, 1), lambda i: (0, 0))],
        out_shape=[jax.ShapeDtypeStruct((N_RES, D), F32),
                   jax.ShapeDtypeStruct((N_RES, D), F32),
                   jax.ShapeDtypeStruct((N_RES, 4), F32),
                   jax.ShapeDtypeStruct((D_EDGE, 1), F32)],
    )(resources, W_self, W_res, a_mat, b_mat, W_op, att_op)

    s2f = s2col.reshape(-1)
    scal_flat = scal.reshape(-1)

    vkron = jnp.kron(jnp.eye(128, dtype=F32), v2)
    c2d = pl.pallas_call(
        _t3_body,
        grid=(5,),
        in_specs=[pl.BlockSpec((512, 128 * D_EDGE), lambda i: (i, 0)),
                  pl.BlockSpec((128 * D_EDGE, 128), lambda i: (0, 0))],
        out_specs=pl.BlockSpec((512, 128), lambda i: (i, 0)),
        out_shape=jax.ShapeDtypeStruct((2560, 128), F32),
    )(attr2, vkron)
    c_flat = c2d.reshape(-1)

    lneed, lsame = _logits_kernel(scal_flat, s2f, c_flat, ns, nd, ss, sd)

    m = pl.pallas_call(
        _max_body,
        out_shape=jax.ShapeDtypeStruct((1, 16), F32),
    )(lneed.reshape(E // 128, 128), lsame.reshape(E // 128, 128), scal)

    accp, zpf = _heavy_kernel(op_proj, rp, lneed, lsame, ns, nd, ss, sd, m)
    a16p = _attr_kernel(lneed, nd, attr_flat, m)
    zp = zpf.reshape(NW, 16)

    emb = pl.pallas_call(
        _fin_body,
        grid=(N_RES // BLK2,),
        in_specs=[pl.BlockSpec((BLK2, D), lambda i: (i, 0)),
                  pl.BlockSpec((BLK2, 4), lambda i: (i, 0)),
                  pl.BlockSpec((BLK2, D), lambda i: (i, 0)),
                  pl.BlockSpec((BLK2, D), lambda i: (i + 10, 0)),
                  pl.BlockSpec((BLK2, D), lambda i: (i, 0)),
                  pl.BlockSpec((BLK2, D), lambda i: (i + 10, 0)),
                  pl.BlockSpec((D_OPF + D_EDGE, D), lambda i: (0, 0)),
                  pl.BlockSpec((1, 16), lambda i: (0, 0)),
                  pl.BlockSpec((NW, 16), lambda i: (0, 0)),
                  pl.BlockSpec((N_RES, 4), lambda i: (0, 0))],
        out_specs=pl.BlockSpec((BLK2, D), lambda i: (i, 0)),
        out_shape=jax.ShapeDtypeStruct((N_RES, D), F32),
    )(sr, scal, accp, accp, a16p, a16p, W_op, m, zp, scal)

    return emb
